# Initial kernel scaffold; baseline (speedup 1.0000x reference)
import functools

import jax
import jax.numpy as jnp
from jax import lax
from jax.experimental import pallas as pl
from jax.experimental.pallas import tpu as pltpu
from jax.experimental.pallas import tpu_sc as plsc

N = 10000
E = 320000
D_IN = 128
D_H = 256
D_OUT = 128

NC = 2
NS = 16
NW = NC * NS
EPW = E // NW
CHUNK = 80
NCHUNK = EPW // CHUNK
ROWS_PER_TILE = N // NS


def _sc_agg_body(x_hbm, src_hbm, dst_hbm, w_hbm, out_hbm,
                 src_v, dst_v, w_v, rows_v, agg_sh, sem):
    c = lax.axis_index("c")
    s = lax.axis_index("s")
    wid = c * NS + s

    pltpu.sync_copy(src_hbm.at[wid], src_v)
    pltpu.sync_copy(dst_hbm.at[wid], dst_v)
    pltpu.sync_copy(w_hbm.at[wid], w_v)

    zv = jnp.zeros((16,), jnp.float32)

    def zero_body(i, carry):
        for j in range(D_IN // 16):
            rows_v[i, pl.ds(j * 16, 16)] = zv
        return carry

    lax.fori_loop(0, CHUNK, zero_body, 0)

    base = s * ROWS_PER_TILE
    nfull = ROWS_PER_TILE // CHUNK
    rem = ROWS_PER_TILE - nfull * CHUNK
    for kk in range(nfull):
        pltpu.sync_copy(rows_v, agg_sh.at[pl.ds(base + kk * CHUNK, CHUNK)])
    if rem:
        pltpu.sync_copy(rows_v.at[pl.ds(0, rem)],
                        agg_sh.at[pl.ds(base + nfull * CHUNK, rem)])
    plsc.subcore_barrier()

    def chunk_body(k, carry):
        pltpu.async_copy(x_hbm.at[src_v.at[pl.ds(k * CHUNK, CHUNK)]],
                         rows_v, sem).wait()

        def edge_body(e, carry2):
            wsp = plsc.load_gather(
                w_v, [jnp.full((16,), k * CHUNK + e, jnp.int32)])
            for j in range(D_IN // 16):
                sl = pl.ds(j * 16, 16)
                rows_v[e, sl] = rows_v[e, sl] * wsp
            return carry2

        lax.fori_loop(0, CHUNK, edge_body, 0)
        pltpu.sync_copy(rows_v, agg_sh.at[dst_v.at[k]], add=True)
        return carry

    lax.fori_loop(0, NCHUNK, chunk_body, 0)
    plsc.subcore_barrier()

    pltpu.sync_copy(agg_sh.at[pl.ds(base, ROWS_PER_TILE)],
                    out_hbm.at[c, pl.ds(base, ROWS_PER_TILE)])


_sc_agg = pl.kernel(
    _sc_agg_body,
    out_type=jax.ShapeDtypeStruct((NC, N, D_IN), jnp.float32),
    mesh=plsc.VectorSubcoreMesh(core_axis_name="c", subcore_axis_name="s"),
    scratch_types=[
        pltpu.VMEM((EPW,), jnp.int32),
        pltpu.VMEM((NCHUNK, CHUNK), jnp.int32),
        pltpu.VMEM((EPW,), jnp.float32),
        pltpu.VMEM((CHUNK, D_IN), jnp.float32),
        pltpu.VMEM_SHARED((N, D_IN), jnp.float32),
        pltpu.SemaphoreType.DMA,
    ],
)


def _mlp_body(a_ref, x_ref, wrel_ref, wroot_ref, wh_ref, wout_ref,
              brel_ref, bh_ref, bout_ref, o_ref):
    agg = a_ref[0] + a_ref[1]
    h = (jnp.dot(agg, wrel_ref[...], preferred_element_type=jnp.float32)
         + jnp.dot(x_ref[...], wroot_ref[...],
                   preferred_element_type=jnp.float32)
         + brel_ref[...])
    h = jnp.maximum(h, 0.0)
    h2 = jnp.dot(h, wh_ref[...], preferred_element_type=jnp.float32) + bh_ref[...]
    h2 = jnp.maximum(h2, 0.0) + jnp.log1p(jnp.exp(-jnp.abs(h2)))
    o_ref[...] = (jnp.dot(h2, wout_ref[...], preferred_element_type=jnp.float32)
                  + bout_ref[...])


_BLK = 1000


def _mlp(agg2, x, wrel_t, wroot_t, wh_t, wout_t, brel, bh, bout):
    grid = (N // _BLK,)
    return pl.pallas_call(
        _mlp_body,
        grid=grid,
        in_specs=[
            pl.BlockSpec((NC, _BLK, D_IN), lambda i: (0, i, 0)),
            pl.BlockSpec((_BLK, D_IN), lambda i: (i, 0)),
            pl.BlockSpec((D_IN, D_H), lambda i: (0, 0)),
            pl.BlockSpec((D_IN, D_H), lambda i: (0, 0)),
            pl.BlockSpec((D_H, D_H), lambda i: (0, 0)),
            pl.BlockSpec((D_H, D_OUT), lambda i: (0, 0)),
            pl.BlockSpec((1, D_H), lambda i: (0, 0)),
            pl.BlockSpec((1, D_H), lambda i: (0, 0)),
            pl.BlockSpec((1, D_OUT), lambda i: (0, 0)),
        ],
        out_specs=pl.BlockSpec((_BLK, D_OUT), lambda i: (i, 0)),
        out_shape=jax.ShapeDtypeStruct((N, D_OUT), jnp.float32),
    )(agg2, x, wrel_t, wroot_t, wh_t, wout_t, brel, bh, bout)


def kernel(feature_data, edge_info, edge_weights, W_rel, b_rel, W_root,
           W_h, b_h, W_out, b_out):
    src = edge_info[0].astype(jnp.int32).reshape(NW, EPW)
    dst = edge_info[1].astype(jnp.int32).reshape(NW, NCHUNK, CHUNK)
    w = edge_weights.reshape(NW, EPW)
    agg2 = _sc_agg(feature_data, src, dst, w)
    return _mlp(agg2, feature_data, W_rel.T, W_root.T, W_h.T, W_out.T,
                b_rel[None, :], b_h[None, :], b_out[None, :])


# speedup vs baseline: 5.2653x; 5.2653x over previous
import functools

import jax
import jax.numpy as jnp
from jax import lax
from jax.experimental import pallas as pl
from jax.experimental.pallas import tpu as pltpu
from jax.experimental.pallas import tpu_sc as plsc

N = 10000
E = 320000
D_IN = 128
D_H = 256
D_OUT = 128

NC = 2
NS = 16
NW = NC * NS
EPW = E // NW
CHUNK = 80
NCHUNK = EPW // CHUNK
OWN_TILES = 10
ROWS_PER_TILE = N // OWN_TILES


def _sc_agg_body(x_hbm, src_hbm, dst_hbm, w_hbm, out_hbm,
                 src_v, dst_v, w_v, rows_v, agg_sh, sem):
    c = lax.axis_index("c")
    s = lax.axis_index("s")
    wid = c * NS + s

    pltpu.sync_copy(src_hbm.at[wid], src_v)
    pltpu.sync_copy(dst_hbm.at[wid], dst_v)
    pltpu.sync_copy(w_hbm.at[wid], w_v)

    zv = jnp.zeros((16,), jnp.float32)

    def zero_body(i, carry):
        for j in range(D_IN // 16):
            rows_v[i, pl.ds(j * 16, 16)] = zv
        return carry

    lax.fori_loop(0, CHUNK, zero_body, 0)

    base = s * ROWS_PER_TILE
    nfull = ROWS_PER_TILE // CHUNK
    rem = ROWS_PER_TILE - nfull * CHUNK

    @pl.when(s < OWN_TILES)
    def _zero_slice():
        for kk in range(nfull):
            pltpu.sync_copy(rows_v, agg_sh.at[pl.ds(base + kk * CHUNK, CHUNK)])
        if rem:
            pltpu.sync_copy(rows_v.at[pl.ds(0, rem)],
                            agg_sh.at[pl.ds(base + nfull * CHUNK, rem)])

    plsc.subcore_barrier()

    def chunk_body(k, carry):
        pltpu.async_copy(x_hbm.at[src_v.at[pl.ds(k * CHUNK, CHUNK)]],
                         rows_v, sem).wait()

        def edge_body(e, carry2):
            wsp = plsc.load_gather(
                w_v, [jnp.full((16,), k * CHUNK + e, jnp.int32)])
            for j in range(D_IN // 16):
                sl = pl.ds(j * 16, 16)
                rows_v[e, sl] = rows_v[e, sl] * wsp
            return carry2

        lax.fori_loop(0, CHUNK, edge_body, 0)
        pltpu.sync_copy(rows_v, agg_sh.at[dst_v.at[k]], add=True)
        return carry

    lax.fori_loop(0, NCHUNK, chunk_body, 0)
    plsc.subcore_barrier()

    @pl.when(s < OWN_TILES)
    def _writeout():
        pltpu.sync_copy(agg_sh.at[pl.ds(base, ROWS_PER_TILE)],
                        out_hbm.at[c, pl.ds(base, ROWS_PER_TILE)])


_sc_agg = pl.kernel(
    _sc_agg_body,
    out_type=jax.ShapeDtypeStruct((NC, N, D_IN), jnp.float32),
    mesh=plsc.VectorSubcoreMesh(core_axis_name="c", subcore_axis_name="s"),
    compiler_params=pltpu.CompilerParams(needs_layout_passes=False),
    scratch_types=[
        pltpu.VMEM((EPW,), jnp.int32),
        pltpu.VMEM((NCHUNK, CHUNK), jnp.int32),
        pltpu.VMEM((EPW,), jnp.float32),
        pltpu.VMEM((CHUNK, D_IN), jnp.float32),
        pltpu.VMEM_SHARED((N, D_IN), jnp.float32),
        pltpu.SemaphoreType.DMA,
    ],
)


def _mlp_body(a_ref, x_ref, wrel_ref, wroot_ref, wh_ref, wout_ref,
              brel_ref, bh_ref, bout_ref, o_ref):
    agg = a_ref[0] + a_ref[1]
    h = (jnp.dot(agg, wrel_ref[...], preferred_element_type=jnp.float32)
         + jnp.dot(x_ref[...], wroot_ref[...],
                   preferred_element_type=jnp.float32)
         + brel_ref[...])
    h = jnp.maximum(h, 0.0)
    h2 = jnp.dot(h, wh_ref[...], preferred_element_type=jnp.float32) + bh_ref[...]
    h2 = jnp.maximum(h2, 0.0) + jnp.log1p(jnp.exp(-jnp.abs(h2)))
    o_ref[...] = (jnp.dot(h2, wout_ref[...], preferred_element_type=jnp.float32)
                  + bout_ref[...])


_BLK = 1000


def _mlp(agg2, x, wrel_t, wroot_t, wh_t, wout_t, brel, bh, bout):
    grid = (N // _BLK,)
    return pl.pallas_call(
        _mlp_body,
        grid=grid,
        in_specs=[
            pl.BlockSpec((NC, _BLK, D_IN), lambda i: (0, i, 0)),
            pl.BlockSpec((_BLK, D_IN), lambda i: (i, 0)),
            pl.BlockSpec((D_IN, D_H), lambda i: (0, 0)),
            pl.BlockSpec((D_IN, D_H), lambda i: (0, 0)),
            pl.BlockSpec((D_H, D_H), lambda i: (0, 0)),
            pl.BlockSpec((D_H, D_OUT), lambda i: (0, 0)),
            pl.BlockSpec((1, D_H), lambda i: (0, 0)),
            pl.BlockSpec((1, D_H), lambda i: (0, 0)),
            pl.BlockSpec((1, D_OUT), lambda i: (0, 0)),
        ],
        out_specs=pl.BlockSpec((_BLK, D_OUT), lambda i: (i, 0)),
        out_shape=jax.ShapeDtypeStruct((N, D_OUT), jnp.float32),
    )(agg2, x, wrel_t, wroot_t, wh_t, wout_t, brel, bh, bout)


def kernel(feature_data, edge_info, edge_weights, W_rel, b_rel, W_root,
           W_h, b_h, W_out, b_out):
    src = edge_info[0].astype(jnp.int32).reshape(NW, EPW)
    dst = edge_info[1].astype(jnp.int32).reshape(NW, NCHUNK, CHUNK)
    w = edge_weights.reshape(NW, EPW)
    agg2 = _sc_agg(feature_data, src, dst, w)
    return _mlp(agg2, feature_data, W_rel.T, W_root.T, W_h.T, W_out.T,
                b_rel[None, :], b_h[None, :], b_out[None, :])


# speedup vs baseline: 6.8182x; 1.2949x over previous
import jax
import jax.numpy as jnp
from jax import lax
from jax.experimental import pallas as pl
from jax.experimental.pallas import tpu as pltpu
from jax.experimental.pallas import tpu_sc as plsc

N = 10000
E = 320000
D_IN = 128
D_H = 256
D_OUT = 128

NC = 2
NS = 16
NW = NC * NS
EPW = E // NW
CHUNK = 120
NCHUNK = 84
EPW_P = NCHUNK * CHUNK
GROUPS = NCHUNK // 6
NBUF = 3
NMETA = 6
OWN_TILES = 10
ROWS_PER_TILE = N // OWN_TILES


def _sc_agg_body(x_hbm, meta_hbm, out_hbm,
                 m0, m1, m2, m3, m4, m5, rows0, rows1, rows2, agg_sh,
                 gsem0, gsem1, gsem2, ssem0, ssem1, ssem2,
                 msem0, msem1, msem2, msem3, msem4, msem5):
    rows = (rows0, rows1, rows2)
    meta = (m0, m1, m2, m3, m4, m5)
    gsem = (gsem0, gsem1, gsem2)
    ssem = (ssem0, ssem1, ssem2)
    msem = (msem0, msem1, msem2, msem3, msem4, msem5)
    c = lax.axis_index("c")
    s = lax.axis_index("s")
    wid = c * NS + s

    def m_desc(k, t):
        return pltpu.make_async_copy(meta_hbm.at[wid, k], meta[t], msem[t])

    def g_desc(k, b, t):
        return pltpu.make_async_copy(
            x_hbm.at[meta[t].at[0]], rows[b], gsem[b])

    def s_desc(b, t):
        return pltpu.make_async_copy(rows[b], agg_sh.at[meta[t].at[1]],
                                     ssem[b])

    zv = jnp.zeros((16,), jnp.float32)

    def zero_body(i, carry):
        for j in range(D_IN // 16):
            rows0[i, pl.ds(j * 16, 16)] = zv
        return carry

    lax.fori_loop(0, CHUNK, zero_body, 0)

    base = s * ROWS_PER_TILE
    nfull = ROWS_PER_TILE // CHUNK
    rem = ROWS_PER_TILE - nfull * CHUNK

    @pl.when(s < OWN_TILES)
    def _zero_slice():
        for kk in range(nfull):
            pltpu.sync_copy(rows0, agg_sh.at[pl.ds(base + kk * CHUNK, CHUNK)])
        if rem:
            pltpu.sync_copy(rows0.at[pl.ds(0, rem)],
                            agg_sh.at[pl.ds(base + nfull * CHUNK, rem)])

    plsc.subcore_barrier()

    for t in range(5):
        m_desc(t, t).start()
    m_desc(0, 0).wait()
    m_desc(1, 1).wait()
    g_desc(0, 0, 0).start()
    g_desc(1, 1, 1).start()

    def group_body(j, carry):
        for i in range(6):
            k = j * 6 + i
            b = i % 3
            t = i
            b2 = (i + 2) % 3
            t2 = (i + 2) % 6
            t5 = (i + 5) % 6
            g_desc(k, b, t).wait()

            @plsc.parallel_loop(0, CHUNK, step=1, unroll=4)
            def _scale(e):
                wbits = plsc.load_gather(
                    meta[t], [jnp.full((16,), 2, jnp.int32),
                              jnp.full((16,), e, jnp.int32)])
                wsp = plsc.bitcast(wbits, jnp.float32)
                for jj in range(D_IN // 16):
                    sl = pl.ds(jj * 16, 16)
                    rows[b][e, sl] = rows[b][e, sl] * wsp

            if i == 0:
                @pl.when(j > 0)
                def _wait_prev():
                    s_desc(b2, t5).wait()
            else:
                s_desc(b2, t5).wait()

            @pl.when(k + 5 < NCHUNK)
            def _next_meta():
                m_desc(k + 5, t5).start()

            @pl.when(k + 2 < NCHUNK)
            def _next_gather():
                m_desc(k + 2, t2).wait()
                g_desc(k + 2, b2, t2).start()

            pltpu.async_copy(rows[b], agg_sh.at[meta[t].at[1]], ssem[b],
                             add=True)
        return carry

    lax.fori_loop(0, GROUPS, group_body, 0)
    s_desc(2, 5).wait()
    plsc.subcore_barrier()

    @pl.when(s < OWN_TILES)
    def _writeout():
        pltpu.sync_copy(agg_sh.at[pl.ds(base, ROWS_PER_TILE)],
                        out_hbm.at[c, pl.ds(base, ROWS_PER_TILE)])


_sc_agg = pl.kernel(
    _sc_agg_body,
    out_type=jax.ShapeDtypeStruct((NC, N, D_IN), jnp.float32),
    mesh=plsc.VectorSubcoreMesh(core_axis_name="c", subcore_axis_name="s"),
    compiler_params=pltpu.CompilerParams(needs_layout_passes=False),
    scratch_types=(
        [pltpu.VMEM((3, CHUNK), jnp.int32) for _ in range(NMETA)]
        + [pltpu.VMEM((CHUNK, D_IN), jnp.float32) for _ in range(NBUF)]
        + [pltpu.VMEM_SHARED((N, D_IN), jnp.float32)]
        + [pltpu.SemaphoreType.DMA for _ in range(NBUF * 2 + NMETA)]
    ),
)


def _mlp_body(a_ref, x_ref, wrel_ref, wroot_ref, wh_ref, wout_ref,
              brel_ref, bh_ref, bout_ref, o_ref):
    agg = a_ref[0] + a_ref[1]
    h = (jnp.dot(agg, wrel_ref[...], preferred_element_type=jnp.float32)
         + jnp.dot(x_ref[...], wroot_ref[...],
                   preferred_element_type=jnp.float32)
         + brel_ref[...])
    h = jnp.maximum(h, 0.0)
    h2 = jnp.dot(h, wh_ref[...], preferred_element_type=jnp.float32) + bh_ref[...]
    h2 = jnp.maximum(h2, 0.0) + jnp.log1p(jnp.exp(-jnp.abs(h2)))
    o_ref[...] = (jnp.dot(h2, wout_ref[...], preferred_element_type=jnp.float32)
                  + bout_ref[...])


_BLK = 1000


def _mlp(agg2, x, wrel_t, wroot_t, wh_t, wout_t, brel, bh, bout):
    grid = (N // _BLK,)
    return pl.pallas_call(
        _mlp_body,
        grid=grid,
        in_specs=[
            pl.BlockSpec((NC, _BLK, D_IN), lambda i: (0, i, 0)),
            pl.BlockSpec((_BLK, D_IN), lambda i: (i, 0)),
            pl.BlockSpec((D_IN, D_H), lambda i: (0, 0)),
            pl.BlockSpec((D_IN, D_H), lambda i: (0, 0)),
            pl.BlockSpec((D_H, D_H), lambda i: (0, 0)),
            pl.BlockSpec((D_H, D_OUT), lambda i: (0, 0)),
            pl.BlockSpec((1, D_H), lambda i: (0, 0)),
            pl.BlockSpec((1, D_H), lambda i: (0, 0)),
            pl.BlockSpec((1, D_OUT), lambda i: (0, 0)),
        ],
        out_specs=pl.BlockSpec((_BLK, D_OUT), lambda i: (i, 0)),
        out_shape=jax.ShapeDtypeStruct((N, D_OUT), jnp.float32),
    )(agg2, x, wrel_t, wroot_t, wh_t, wout_t, brel, bh, bout)


def kernel(feature_data, edge_info, edge_weights, W_rel, b_rel, W_root,
           W_h, b_h, W_out, b_out):
    pad = EPW_P - EPW
    src = edge_info[0].astype(jnp.int32).reshape(NW, EPW)
    src = jnp.pad(src, ((0, 0), (0, pad))).reshape(NW, NCHUNK, CHUNK)
    dst = edge_info[1].astype(jnp.int32).reshape(NW, EPW)
    dst = jnp.pad(dst, ((0, 0), (0, pad))).reshape(NW, NCHUNK, CHUNK)
    wbits = jax.lax.bitcast_convert_type(edge_weights, jnp.int32)
    wbits = wbits.reshape(NW, EPW)
    wbits = jnp.pad(wbits, ((0, 0), (0, pad))).reshape(NW, NCHUNK, CHUNK)
    meta = jnp.stack([src, dst, wbits], axis=2)
    agg2 = _sc_agg(feature_data, meta)
    return _mlp(agg2, feature_data, W_rel.T, W_root.T, W_h.T, W_out.T,
                b_rel[None, :], b_h[None, :], b_out[None, :])


# speedup vs baseline: 11.0892x; 1.6264x over previous
import jax
import jax.numpy as jnp
from jax import lax
from jax.experimental import pallas as pl
from jax.experimental.pallas import tpu as pltpu
from jax.experimental.pallas import tpu_sc as plsc

N = 10000
E = 320000
D_IN = 128
D_H = 256
D_OUT = 128

NC = 2
NS = 16
NW = NC * NS
EPW = E // NW
CHUNK = 80
NCHUNK = EPW // CHUNK
NBUF = 3
NMETA = 6
GROUPS = NCHUNK // 6
TAIL = NCHUNK - GROUPS * 6
OWN_TILES = 10
ROWS_PER_TILE = N // OWN_TILES


def _sc_agg_body(x_hbm, src_hbm, dst_hbm, w_hbm, out_hbm,
                 s0, s1, s2, s3, s4, s5, d0, d1, d2, d3, d4, d5,
                 w0, w1, w2, w3, w4, w5, rows0, rows1, rows2, agg_sh,
                 gsem0, gsem1, gsem2, ssem0, ssem1, ssem2,
                 msem0, msem1, msem2, msem3, msem4, msem5):
    rows = (rows0, rows1, rows2)
    msrc = (s0, s1, s2, s3, s4, s5)
    mdst = (d0, d1, d2, d3, d4, d5)
    mw = (w0, w1, w2, w3, w4, w5)
    gsem = (gsem0, gsem1, gsem2)
    ssem = (ssem0, ssem1, ssem2)
    msem = (msem0, msem1, msem2, msem3, msem4, msem5)
    c = lax.axis_index("c")
    s = lax.axis_index("s")
    wid = c * NS + s
    ebase = wid * EPW

    def m_descs(k, t):
        sl = pl.ds(ebase + k * CHUNK, CHUNK)
        return (pltpu.make_async_copy(src_hbm.at[sl], msrc[t], msem[t]),
                pltpu.make_async_copy(dst_hbm.at[sl], mdst[t], msem[t]),
                pltpu.make_async_copy(w_hbm.at[sl], mw[t], msem[t]))

    def m_start(k, t):
        for d in m_descs(k, t):
            d.start()

    def m_wait(k, t):
        for d in m_descs(k, t):
            d.wait()

    def g_desc(b, t):
        return pltpu.make_async_copy(x_hbm.at[msrc[t]], rows[b], gsem[b])

    def s_desc(b, t):
        return pltpu.make_async_copy(rows[b], agg_sh.at[mdst[t]], ssem[b])

    def scale(b, t):
        @plsc.parallel_loop(0, CHUNK, step=1, unroll=8)
        def _scale(e):
            wsp = plsc.load_gather(mw[t], [jnp.full((16,), e, jnp.int32)])
            for jj in range(D_IN // 16):
                sl = pl.ds(jj * 16, 16)
                rows[b][e, sl] = rows[b][e, sl] * wsp

    for t in range(5):
        m_start(t, t)
    m_wait(0, 0)
    m_wait(1, 1)
    g_desc(0, 0).start()
    g_desc(1, 1).start()

    zv = jnp.zeros((16,), jnp.float32)

    def zero_body(i, carry):
        for j in range(D_IN // 16):
            rows2[i, pl.ds(j * 16, 16)] = zv
        return carry

    lax.fori_loop(0, CHUNK, zero_body, 0)

    base = s * ROWS_PER_TILE

    nfull = ROWS_PER_TILE // CHUNK
    rem = ROWS_PER_TILE - nfull * CHUNK

    @pl.when(s < OWN_TILES)
    def _zero_slice():
        for kk in range(nfull):
            pltpu.sync_copy(rows2,
                            agg_sh.at[pl.ds(base + kk * CHUNK, CHUNK)])
        if rem:
            pltpu.sync_copy(rows2.at[pl.ds(0, rem)],
                            agg_sh.at[pl.ds(base + nfull * CHUNK, rem)])

    plsc.subcore_barrier()

    def group_body(j, carry):
        for i in range(6):
            k = j * 6 + i
            b = i % 3
            t = i
            b2 = (i + 2) % 3
            t2 = (i + 2) % 6
            t5 = (i + 5) % 6
            g_desc(b, t).wait()
            scale(b, t)

            if i == 0:
                @pl.when(j > 0)
                def _wait_prev():
                    s_desc(b2, t5).wait()
            else:
                s_desc(b2, t5).wait()

            @pl.when(k + 5 < NCHUNK)
            def _next_meta():
                m_start(k + 5, t5)

            @pl.when(k + 2 < NCHUNK)
            def _next_gather():
                m_wait(k + 2, t2)
                g_desc(b2, t2).start()

            pltpu.async_copy(rows[b], agg_sh.at[mdst[t]], ssem[b],
                             add=True)
        return carry

    lax.fori_loop(0, GROUPS, group_body, 0)

    for r in range(TAIL):
        k = GROUPS * 6 + r
        b = k % 3
        t = k % 6
        g_desc(b, t).wait()
        scale(b, t)
        s_desc((k - 1) % 3, (k - 1) % 6).wait()
        if k + 5 < NCHUNK:
            m_start(k + 5, (k + 5) % 6)
        if k + 2 < NCHUNK:
            m_wait(k + 2, (k + 2) % 6)
            g_desc((k + 2) % 3, (k + 2) % 6).start()
        pltpu.async_copy(rows[b], agg_sh.at[mdst[t]], ssem[b], add=True)
    s_desc((NCHUNK - 1) % 3, (NCHUNK - 1) % 6).wait()
    plsc.subcore_barrier()

    @pl.when(s < OWN_TILES)
    def _writeout():
        pltpu.sync_copy(agg_sh.at[pl.ds(base, ROWS_PER_TILE)],
                        out_hbm.at[c, pl.ds(base, ROWS_PER_TILE)])


_sc_agg = pl.kernel(
    _sc_agg_body,
    out_type=jax.ShapeDtypeStruct((NC, N, D_IN), jnp.float32),
    mesh=plsc.VectorSubcoreMesh(core_axis_name="c", subcore_axis_name="s"),
    compiler_params=pltpu.CompilerParams(needs_layout_passes=False),
    scratch_types=(
        [pltpu.VMEM((CHUNK,), jnp.int32) for _ in range(NMETA)]
        + [pltpu.VMEM((CHUNK,), jnp.int32) for _ in range(NMETA)]
        + [pltpu.VMEM((CHUNK,), jnp.float32) for _ in range(NMETA)]
        + [pltpu.VMEM((CHUNK, D_IN), jnp.float32) for _ in range(NBUF)]
        + [pltpu.VMEM_SHARED((N, D_IN), jnp.float32)]
        + [pltpu.SemaphoreType.DMA for _ in range(NBUF * 2 + NMETA)]
    ),
)


def _mlp_body(a_ref, x_ref, wrel_ref, wroot_ref, wh_ref, wout_ref,
              brel_ref, bh_ref, bout_ref, o_ref):
    agg = a_ref[0] + a_ref[1]
    h = (jnp.dot(agg, wrel_ref[...], preferred_element_type=jnp.float32)
         + jnp.dot(x_ref[...], wroot_ref[...],
                   preferred_element_type=jnp.float32)
         + brel_ref[...])
    h = jnp.maximum(h, 0.0)
    h2 = jnp.dot(h, wh_ref[...], preferred_element_type=jnp.float32) + bh_ref[...]
    h2 = jnp.maximum(h2, 0.0) + jnp.log1p(jnp.exp(-jnp.abs(h2)))
    o_ref[...] = (jnp.dot(h2, wout_ref[...], preferred_element_type=jnp.float32)
                  + bout_ref[...])


_BLK = 1000


def _mlp(agg2, x, wrel_t, wroot_t, wh_t, wout_t, brel, bh, bout):
    grid = (N // _BLK,)
    return pl.pallas_call(
        _mlp_body,
        grid=grid,
        in_specs=[
            pl.BlockSpec((NC, _BLK, D_IN), lambda i: (0, i, 0)),
            pl.BlockSpec((_BLK, D_IN), lambda i: (i, 0)),
            pl.BlockSpec((D_IN, D_H), lambda i: (0, 0)),
            pl.BlockSpec((D_IN, D_H), lambda i: (0, 0)),
            pl.BlockSpec((D_H, D_H), lambda i: (0, 0)),
            pl.BlockSpec((D_H, D_OUT), lambda i: (0, 0)),
            pl.BlockSpec((1, D_H), lambda i: (0, 0)),
            pl.BlockSpec((1, D_H), lambda i: (0, 0)),
            pl.BlockSpec((1, D_OUT), lambda i: (0, 0)),
        ],
        out_specs=pl.BlockSpec((_BLK, D_OUT), lambda i: (i, 0)),
        out_shape=jax.ShapeDtypeStruct((N, D_OUT), jnp.float32),
    )(agg2, x, wrel_t, wroot_t, wh_t, wout_t, brel, bh, bout)


def kernel(feature_data, edge_info, edge_weights, W_rel, b_rel, W_root,
           W_h, b_h, W_out, b_out):
    ei = edge_info.astype(jnp.int32)
    agg2 = _sc_agg(feature_data, ei[0], ei[1], edge_weights)
    return _mlp(agg2, feature_data, W_rel.T, W_root.T, W_h.T, W_out.T,
                b_rel[None, :], b_h[None, :], b_out[None, :])


# speedup vs baseline: 11.2126x; 1.0111x over previous
import jax
import jax.numpy as jnp
from jax import lax
from jax.experimental import pallas as pl
from jax.experimental.pallas import tpu as pltpu
from jax.experimental.pallas import tpu_sc as plsc

N = 10000
E = 320000
D_IN = 128
D_H = 256
D_OUT = 128

NC = 2
NS = 16
NW = NC * NS
EPW = E // NW
CHUNK = 80
NCHUNK = EPW // CHUNK
NBUF = 3
NMETA = 6
GROUPS = NCHUNK // 6
TAIL = NCHUNK - GROUPS * 6
ROWS_A = 624
ROWS_B = 632
SPLIT_TILE = 14


def _sc_agg_body(x_hbm, src_hbm, dst_hbm, w_hbm, out_hbm,
                 s0, s1, s2, s3, s4, s5, d0, d1, d2, d3, d4, d5,
                 w0, w1, w2, w3, w4, w5, rows0, rows1, rows2, agg_sh,
                 gsem0, gsem1, gsem2, ssem0, ssem1, ssem2,
                 msem0, msem1, msem2, msem3, msem4, msem5):
    rows = (rows0, rows1, rows2)
    msrc = (s0, s1, s2, s3, s4, s5)
    mdst = (d0, d1, d2, d3, d4, d5)
    mw = (w0, w1, w2, w3, w4, w5)
    gsem = (gsem0, gsem1, gsem2)
    ssem = (ssem0, ssem1, ssem2)
    msem = (msem0, msem1, msem2, msem3, msem4, msem5)
    c = lax.axis_index("c")
    s = lax.axis_index("s")
    wid = c * NS + s
    ebase = wid * EPW

    def m_descs(k, t):
        sl = pl.ds(ebase + k * CHUNK, CHUNK)
        return (pltpu.make_async_copy(src_hbm.at[sl], msrc[t], msem[t]),
                pltpu.make_async_copy(dst_hbm.at[sl], mdst[t], msem[t]),
                pltpu.make_async_copy(w_hbm.at[sl], mw[t], msem[t]))

    def m_start(k, t):
        for d in m_descs(k, t):
            d.start()

    def m_wait(k, t):
        for d in m_descs(k, t):
            d.wait()

    def g_desc(b, t):
        return pltpu.make_async_copy(x_hbm.at[msrc[t]], rows[b], gsem[b])

    def s_desc(b, t):
        return pltpu.make_async_copy(rows[b], agg_sh.at[mdst[t]], ssem[b])

    def scale(b, t):
        @plsc.parallel_loop(0, CHUNK, step=1, unroll=8)
        def _scale(e):
            wsp = plsc.load_gather(mw[t], [jnp.full((16,), e, jnp.int32)])
            for jj in range(D_IN // 16):
                sl = pl.ds(jj * 16, 16)
                rows[b][e, sl] = rows[b][e, sl] * wsp

    for t in range(5):
        m_start(t, t)
    m_wait(0, 0)
    m_wait(1, 1)
    g_desc(0, 0).start()
    g_desc(1, 1).start()

    zv = jnp.zeros((16,), jnp.float32)

    def zero_body(i, carry):
        for j in range(D_IN // 16):
            rows2[i, pl.ds(j * 16, 16)] = zv
        return carry

    lax.fori_loop(0, CHUNK, zero_body, 0)

    base = jnp.where(s < SPLIT_TILE, s * ROWS_A, s * ROWS_B - 112)

    def _slice_copies(nrows, copy_fn):
        nfull = nrows // CHUNK
        rem = nrows - nfull * CHUNK
        for kk in range(nfull):
            copy_fn(pl.ds(base + kk * CHUNK, CHUNK), CHUNK)
        if rem:
            copy_fn(pl.ds(base + nfull * CHUNK, rem), rem)

    def _zero_copy(dst_sl, nr):
        pltpu.sync_copy(rows2.at[pl.ds(0, nr)], agg_sh.at[dst_sl])

    @pl.when(s < SPLIT_TILE)
    def _zero_slice_a():
        _slice_copies(ROWS_A, _zero_copy)

    @pl.when(s >= SPLIT_TILE)
    def _zero_slice_b():
        _slice_copies(ROWS_B, _zero_copy)

    plsc.subcore_barrier()

    def group_body(j, carry):
        for i in range(6):
            k = j * 6 + i
            b = i % 3
            t = i
            b2 = (i + 2) % 3
            t2 = (i + 2) % 6
            t5 = (i + 5) % 6
            g_desc(b, t).wait()
            scale(b, t)

            if i == 0:
                @pl.when(j > 0)
                def _wait_prev():
                    s_desc(b2, t5).wait()
            else:
                s_desc(b2, t5).wait()

            @pl.when(k + 5 < NCHUNK)
            def _next_meta():
                m_start(k + 5, t5)

            @pl.when(k + 2 < NCHUNK)
            def _next_gather():
                m_wait(k + 2, t2)
                g_desc(b2, t2).start()

            pltpu.async_copy(rows[b], agg_sh.at[mdst[t]], ssem[b],
                             add=True)
        return carry

    lax.fori_loop(0, GROUPS, group_body, 0)

    for r in range(TAIL):
        k = GROUPS * 6 + r
        b = k % 3
        t = k % 6
        g_desc(b, t).wait()
        scale(b, t)
        s_desc((k - 1) % 3, (k - 1) % 6).wait()
        if k + 5 < NCHUNK:
            m_start(k + 5, (k + 5) % 6)
        if k + 2 < NCHUNK:
            m_wait(k + 2, (k + 2) % 6)
            g_desc((k + 2) % 3, (k + 2) % 6).start()
        pltpu.async_copy(rows[b], agg_sh.at[mdst[t]], ssem[b], add=True)
    s_desc((NCHUNK - 1) % 3, (NCHUNK - 1) % 6).wait()
    plsc.subcore_barrier()

    @pl.when(s < SPLIT_TILE)
    def _writeout_a():
        pltpu.sync_copy(agg_sh.at[pl.ds(base, ROWS_A)],
                        out_hbm.at[c, pl.ds(base, ROWS_A)])

    @pl.when(s >= SPLIT_TILE)
    def _writeout_b():
        pltpu.sync_copy(agg_sh.at[pl.ds(base, ROWS_B)],
                        out_hbm.at[c, pl.ds(base, ROWS_B)])


_sc_agg = pl.kernel(
    _sc_agg_body,
    out_type=jax.ShapeDtypeStruct((NC, N, D_IN), jnp.float32),
    mesh=plsc.VectorSubcoreMesh(core_axis_name="c", subcore_axis_name="s"),
    compiler_params=pltpu.CompilerParams(needs_layout_passes=False),
    scratch_types=(
        [pltpu.VMEM((CHUNK,), jnp.int32) for _ in range(NMETA)]
        + [pltpu.VMEM((CHUNK,), jnp.int32) for _ in range(NMETA)]
        + [pltpu.VMEM((CHUNK,), jnp.float32) for _ in range(NMETA)]
        + [pltpu.VMEM((CHUNK, D_IN), jnp.float32) for _ in range(NBUF)]
        + [pltpu.VMEM_SHARED((N, D_IN), jnp.float32)]
        + [pltpu.SemaphoreType.DMA for _ in range(NBUF * 2 + NMETA)]
    ),
)


def _mlp_body(a_ref, x_ref, wrel_ref, wroot_ref, wh_ref, wout_ref,
              brel_ref, bh_ref, bout_ref, o_ref):
    agg = a_ref[0] + a_ref[1]
    h = (jnp.dot(agg, wrel_ref[...], preferred_element_type=jnp.float32)
         + jnp.dot(x_ref[...], wroot_ref[...],
                   preferred_element_type=jnp.float32)
         + brel_ref[...])
    h = jnp.maximum(h, 0.0)
    h2 = jnp.dot(h, wh_ref[...], preferred_element_type=jnp.float32) + bh_ref[...]
    h2 = jnp.maximum(h2, 0.0) + jnp.log1p(jnp.exp(-jnp.abs(h2)))
    o_ref[...] = (jnp.dot(h2, wout_ref[...], preferred_element_type=jnp.float32)
                  + bout_ref[...])


_BLK = 1000


def _mlp(agg2, x, wrel_t, wroot_t, wh_t, wout_t, brel, bh, bout):
    grid = (N // _BLK,)
    return pl.pallas_call(
        _mlp_body,
        grid=grid,
        in_specs=[
            pl.BlockSpec((NC, _BLK, D_IN), lambda i: (0, i, 0)),
            pl.BlockSpec((_BLK, D_IN), lambda i: (i, 0)),
            pl.BlockSpec((D_IN, D_H), lambda i: (0, 0)),
            pl.BlockSpec((D_IN, D_H), lambda i: (0, 0)),
            pl.BlockSpec((D_H, D_H), lambda i: (0, 0)),
            pl.BlockSpec((D_H, D_OUT), lambda i: (0, 0)),
            pl.BlockSpec((1, D_H), lambda i: (0, 0)),
            pl.BlockSpec((1, D_H), lambda i: (0, 0)),
            pl.BlockSpec((1, D_OUT), lambda i: (0, 0)),
        ],
        out_specs=pl.BlockSpec((_BLK, D_OUT), lambda i: (i, 0)),
        out_shape=jax.ShapeDtypeStruct((N, D_OUT), jnp.float32),
    )(agg2, x, wrel_t, wroot_t, wh_t, wout_t, brel, bh, bout)


def kernel(feature_data, edge_info, edge_weights, W_rel, b_rel, W_root,
           W_h, b_h, W_out, b_out):
    ei = edge_info.astype(jnp.int32)
    agg2 = _sc_agg(feature_data, ei[0], ei[1], edge_weights)
    return _mlp(agg2, feature_data, W_rel.T, W_root.T, W_h.T, W_out.T,
                b_rel[None, :], b_h[None, :], b_out[None, :])


# speedup vs baseline: 11.8559x; 1.0574x over previous
import jax
import jax.numpy as jnp
from jax import lax
from jax.experimental import pallas as pl
from jax.experimental.pallas import tpu as pltpu
from jax.experimental.pallas import tpu_sc as plsc

N = 10000
E = 320000
D_IN = 128
D_H = 256
D_OUT = 128

NC = 2
NS = 16
NW = NC * NS
EPW = E // NW
CHUNK = 80
NCHUNK = EPW // CHUNK
NBUF = 4
NMETA = 8
GDEPTH = NBUF - 1
GROUPS = NCHUNK // NMETA
TAIL = NCHUNK - GROUPS * NMETA
ROWS_A = 624
ROWS_B = 632
SPLIT_TILE = 14


def _sc_agg_body(x_hbm, src_hbm, dst_hbm, w_hbm, out_hbm, *scr):
    msrc = scr[0:NMETA]
    mdst = scr[NMETA:2 * NMETA]
    mw = scr[2 * NMETA:3 * NMETA]
    rows = scr[3 * NMETA:3 * NMETA + NBUF]
    agg_sh = scr[3 * NMETA + NBUF]
    sems = scr[3 * NMETA + NBUF + 1:]
    gsem = sems[0:NBUF]
    ssem = sems[NBUF:2 * NBUF]
    msem = sems[2 * NBUF:2 * NBUF + NMETA]
    c = lax.axis_index("c")
    s = lax.axis_index("s")
    wid = c * NS + s
    ebase = wid * EPW

    def m_descs(k, t):
        sl = pl.ds(ebase + k * CHUNK, CHUNK)
        return (pltpu.make_async_copy(src_hbm.at[sl], msrc[t], msem[t]),
                pltpu.make_async_copy(dst_hbm.at[sl], mdst[t], msem[t]),
                pltpu.make_async_copy(w_hbm.at[sl], mw[t], msem[t]))

    def m_start(k, t):
        for d in m_descs(k, t):
            d.start()

    def m_wait(k, t):
        for d in m_descs(k, t):
            d.wait()

    def g_desc(b, t):
        return pltpu.make_async_copy(x_hbm.at[msrc[t]], rows[b], gsem[b])

    def s_desc(b, t):
        return pltpu.make_async_copy(rows[b], agg_sh.at[mdst[t]], ssem[b])

    def scale(b, t):
        @plsc.parallel_loop(0, CHUNK, step=1, unroll=8)
        def _scale(e):
            wsp = plsc.load_gather(mw[t], [jnp.full((16,), e, jnp.int32)])
            for jj in range(D_IN // 16):
                sl = pl.ds(jj * 16, 16)
                rows[b][e, sl] = rows[b][e, sl] * wsp

    zbuf = rows[NBUF - 1]
    for t in range(NMETA - 1):
        m_start(t, t)
    for t in range(GDEPTH):
        m_wait(t, t)
        g_desc(t, t).start()

    zv = jnp.zeros((16,), jnp.float32)

    def zero_body(i, carry):
        for j in range(D_IN // 16):
            zbuf[i, pl.ds(j * 16, 16)] = zv
        return carry

    lax.fori_loop(0, CHUNK, zero_body, 0)

    base = jnp.where(s < SPLIT_TILE, s * ROWS_A, s * ROWS_B - 112)

    def _slice_copies(nrows, copy_fn):
        nfull = nrows // CHUNK
        rem = nrows - nfull * CHUNK
        for kk in range(nfull):
            copy_fn(pl.ds(base + kk * CHUNK, CHUNK), CHUNK)
        if rem:
            copy_fn(pl.ds(base + nfull * CHUNK, rem), rem)

    def _zero_copy(dst_sl, nr):
        pltpu.sync_copy(zbuf.at[pl.ds(0, nr)], agg_sh.at[dst_sl])

    @pl.when(s < SPLIT_TILE)
    def _zero_slice_a():
        _slice_copies(ROWS_A, _zero_copy)

    @pl.when(s >= SPLIT_TILE)
    def _zero_slice_b():
        _slice_copies(ROWS_B, _zero_copy)

    plsc.subcore_barrier()

    def group_body(j, carry):
        for i in range(NMETA):
            k = j * NMETA + i
            b = i % NBUF
            t = i
            bg = (i + GDEPTH) % NBUF
            tg = (i + GDEPTH) % NMETA
            tm = (i + NMETA - 1) % NMETA
            g_desc(b, t).wait()
            scale(b, t)

            if i == 0:
                @pl.when(j > 0)
                def _wait_prev():
                    s_desc((NBUF - 1) % NBUF, NMETA - 1).wait()
            else:
                s_desc((i - 1) % NBUF, i - 1).wait()

            @pl.when(k + NMETA - 1 < NCHUNK)
            def _next_meta():
                m_start(k + NMETA - 1, tm)

            @pl.when(k + GDEPTH < NCHUNK)
            def _next_gather():
                m_wait(k + GDEPTH, tg)
                g_desc(bg, tg).start()

            pltpu.async_copy(rows[b], agg_sh.at[mdst[t]], ssem[b],
                             add=True)
        return carry

    lax.fori_loop(0, GROUPS, group_body, 0)

    for r in range(TAIL):
        k = GROUPS * NMETA + r
        b = k % NBUF
        t = k % NMETA
        g_desc(b, t).wait()
        scale(b, t)
        s_desc((k - 1) % NBUF, (k - 1) % NMETA).wait()
        if k + NMETA - 1 < NCHUNK:
            m_start(k + NMETA - 1, (k + NMETA - 1) % NMETA)
        if k + GDEPTH < NCHUNK:
            m_wait(k + GDEPTH, (k + GDEPTH) % NMETA)
            g_desc((k + GDEPTH) % NBUF, (k + GDEPTH) % NMETA).start()
        pltpu.async_copy(rows[b], agg_sh.at[mdst[t]], ssem[b], add=True)
    s_desc((NCHUNK - 1) % NBUF, (NCHUNK - 1) % NMETA).wait()
    plsc.subcore_barrier()

    @pl.when(s < SPLIT_TILE)
    def _writeout_a():
        pltpu.sync_copy(agg_sh.at[pl.ds(base, ROWS_A)],
                        out_hbm.at[c, pl.ds(base, ROWS_A)])

    @pl.when(s >= SPLIT_TILE)
    def _writeout_b():
        pltpu.sync_copy(agg_sh.at[pl.ds(base, ROWS_B)],
                        out_hbm.at[c, pl.ds(base, ROWS_B)])


_sc_agg = pl.kernel(
    _sc_agg_body,
    out_type=jax.ShapeDtypeStruct((NC, N, D_IN), jnp.float32),
    mesh=plsc.VectorSubcoreMesh(core_axis_name="c", subcore_axis_name="s"),
    compiler_params=pltpu.CompilerParams(needs_layout_passes=False),
    scratch_types=(
        [pltpu.VMEM((CHUNK,), jnp.int32) for _ in range(NMETA)]
        + [pltpu.VMEM((CHUNK,), jnp.int32) for _ in range(NMETA)]
        + [pltpu.VMEM((CHUNK,), jnp.float32) for _ in range(NMETA)]
        + [pltpu.VMEM((CHUNK, D_IN), jnp.float32) for _ in range(NBUF)]
        + [pltpu.VMEM_SHARED((N, D_IN), jnp.float32)]
        + [pltpu.SemaphoreType.DMA for _ in range(NBUF * 2 + NMETA)]
    ),
)


def _mlp_body(a_ref, x_ref, wrel_ref, wroot_ref, wh_ref, wout_ref,
              brel_ref, bh_ref, bout_ref, o_ref):
    agg = a_ref[0] + a_ref[1]
    h = (jnp.dot(agg, wrel_ref[...], preferred_element_type=jnp.float32)
         + jnp.dot(x_ref[...], wroot_ref[...],
                   preferred_element_type=jnp.float32)
         + brel_ref[...])
    h = jnp.maximum(h, 0.0)
    h2 = jnp.dot(h, wh_ref[...], preferred_element_type=jnp.float32) + bh_ref[...]
    h2 = jnp.maximum(h2, 0.0) + jnp.log1p(jnp.exp(-jnp.abs(h2)))
    o_ref[...] = (jnp.dot(h2, wout_ref[...], preferred_element_type=jnp.float32)
                  + bout_ref[...])


_BLK = 1000


def _mlp(agg2, x, wrel_t, wroot_t, wh_t, wout_t, brel, bh, bout):
    grid = (N // _BLK,)
    return pl.pallas_call(
        _mlp_body,
        grid=grid,
        in_specs=[
            pl.BlockSpec((NC, _BLK, D_IN), lambda i: (0, i, 0)),
            pl.BlockSpec((_BLK, D_IN), lambda i: (i, 0)),
            pl.BlockSpec((D_IN, D_H), lambda i: (0, 0)),
            pl.BlockSpec((D_IN, D_H), lambda i: (0, 0)),
            pl.BlockSpec((D_H, D_H), lambda i: (0, 0)),
            pl.BlockSpec((D_H, D_OUT), lambda i: (0, 0)),
            pl.BlockSpec((1, D_H), lambda i: (0, 0)),
            pl.BlockSpec((1, D_H), lambda i: (0, 0)),
            pl.BlockSpec((1, D_OUT), lambda i: (0, 0)),
        ],
        out_specs=pl.BlockSpec((_BLK, D_OUT), lambda i: (i, 0)),
        out_shape=jax.ShapeDtypeStruct((N, D_OUT), jnp.float32),
    )(agg2, x, wrel_t, wroot_t, wh_t, wout_t, brel, bh, bout)


def kernel(feature_data, edge_info, edge_weights, W_rel, b_rel, W_root,
           W_h, b_h, W_out, b_out):
    ei = edge_info.astype(jnp.int32)
    agg2 = _sc_agg(feature_data, ei[0], ei[1], edge_weights)
    return _mlp(agg2, feature_data, W_rel.T, W_root.T, W_h.T, W_out.T,
                b_rel[None, :], b_h[None, :], b_out[None, :])


# speedup vs baseline: 11.9582x; 1.0086x over previous
import jax
import jax.numpy as jnp
from jax import lax
from jax.experimental import pallas as pl
from jax.experimental.pallas import tpu as pltpu
from jax.experimental.pallas import tpu_sc as plsc

N = 10000
E = 320000
D_IN = 128
D_H = 256
D_OUT = 128

NC = 2
NS = 16
NW = NC * NS
EPW = E // NW
CHUNK = 80
NCHUNK = EPW // CHUNK
NBUF = 4
NMETA = 8
GDEPTH = NBUF - 1
GROUPS = NCHUNK // NMETA
TAIL = NCHUNK - GROUPS * NMETA
ROWS_A = 624
ROWS_B = 632
SPLIT_TILE = 14


def _sc_agg_body(x_hbm, src_hbm, dst_hbm, w_hbm, out_hbm, *scr):
    msrc = scr[0:NMETA]
    mdst = scr[NMETA:2 * NMETA]
    mw = scr[2 * NMETA:3 * NMETA]
    rows = scr[3 * NMETA:3 * NMETA + NBUF]
    agg_sh = scr[3 * NMETA + NBUF]
    sems = scr[3 * NMETA + NBUF + 1:]
    gsem = sems[0:NBUF]
    ssem = sems[NBUF:2 * NBUF]
    msem = sems[2 * NBUF:2 * NBUF + NMETA]
    c = lax.axis_index("c")
    s = lax.axis_index("s")
    wid = c * NS + s
    ebase = wid * EPW

    def m_descs(k, t):
        sl = pl.ds(ebase + k * CHUNK, CHUNK)
        return (pltpu.make_async_copy(src_hbm.at[sl], msrc[t], msem[t]),
                pltpu.make_async_copy(dst_hbm.at[sl], mdst[t], msem[t]),
                pltpu.make_async_copy(w_hbm.at[sl], mw[t], msem[t]))

    def m_start(k, t):
        for d in m_descs(k, t):
            d.start()

    def m_wait(k, t):
        for d in m_descs(k, t):
            d.wait()

    def g_desc(b, t):
        return pltpu.make_async_copy(x_hbm.at[msrc[t]], rows[b], gsem[b])

    def s_desc(b, t):
        return pltpu.make_async_copy(rows[b], agg_sh.at[mdst[t]], ssem[b])

    def scale(b, t):
        @plsc.parallel_loop(0, CHUNK, step=1, unroll=8)
        def _scale(e):
            wsp = plsc.load_gather(mw[t], [jnp.full((16,), e, jnp.int32)])
            for jj in range(D_IN // 16):
                sl = pl.ds(jj * 16, 16)
                rows[b][e, sl] = rows[b][e, sl] * wsp

    zbuf = rows[NBUF - 1]
    for t in range(NMETA - 1):
        m_start(t, t)
    for t in range(GDEPTH):
        m_wait(t, t)
        g_desc(t, t).start()

    zv = jnp.zeros((16,), jnp.float32)

    def zero_body(i, carry):
        for j in range(D_IN // 16):
            zbuf[i, pl.ds(j * 16, 16)] = zv
        return carry

    lax.fori_loop(0, CHUNK, zero_body, 0)

    base = jnp.where(s < SPLIT_TILE, s * ROWS_A, s * ROWS_B - 112)

    def _slice_copies(nrows, copy_fn):
        nfull = nrows // CHUNK
        rem = nrows - nfull * CHUNK
        for kk in range(nfull):
            copy_fn(pl.ds(base + kk * CHUNK, CHUNK), CHUNK)
        if rem:
            copy_fn(pl.ds(base + nfull * CHUNK, rem), rem)

    def _zero_copy(dst_sl, nr):
        pltpu.sync_copy(zbuf.at[pl.ds(0, nr)], agg_sh.at[dst_sl])

    @pl.when(s < SPLIT_TILE)
    def _zero_slice_a():
        _slice_copies(ROWS_A, _zero_copy)

    @pl.when(s >= SPLIT_TILE)
    def _zero_slice_b():
        _slice_copies(ROWS_B, _zero_copy)

    plsc.subcore_barrier()

    def group_body(j, carry):
        for i in range(NMETA):
            k = j * NMETA + i
            b = i % NBUF
            t = i
            bg = (i + GDEPTH) % NBUF
            tg = (i + GDEPTH) % NMETA
            tm = (i + NMETA - 1) % NMETA
            g_desc(b, t).wait()
            scale(b, t)

            if i == 0:
                @pl.when(j > 0)
                def _wait_prev():
                    s_desc((NBUF - 1) % NBUF, NMETA - 1).wait()
            else:
                s_desc((i - 1) % NBUF, i - 1).wait()

            @pl.when(k + NMETA - 1 < NCHUNK)
            def _next_meta():
                m_start(k + NMETA - 1, tm)

            @pl.when(k + GDEPTH < NCHUNK)
            def _next_gather():
                m_wait(k + GDEPTH, tg)
                g_desc(bg, tg).start()

            pltpu.async_copy(rows[b], agg_sh.at[mdst[t]], ssem[b],
                             add=True)
        return carry

    lax.fori_loop(0, GROUPS, group_body, 0)

    for r in range(TAIL):
        k = GROUPS * NMETA + r
        b = k % NBUF
        t = k % NMETA
        g_desc(b, t).wait()
        scale(b, t)
        s_desc((k - 1) % NBUF, (k - 1) % NMETA).wait()
        if k + NMETA - 1 < NCHUNK:
            m_start(k + NMETA - 1, (k + NMETA - 1) % NMETA)
        if k + GDEPTH < NCHUNK:
            m_wait(k + GDEPTH, (k + GDEPTH) % NMETA)
            g_desc((k + GDEPTH) % NBUF, (k + GDEPTH) % NMETA).start()
        pltpu.async_copy(rows[b], agg_sh.at[mdst[t]], ssem[b], add=True)
    s_desc((NCHUNK - 1) % NBUF, (NCHUNK - 1) % NMETA).wait()
    plsc.subcore_barrier()

    @pl.when(s < SPLIT_TILE)
    def _writeout_a():
        pltpu.sync_copy(agg_sh.at[pl.ds(base, ROWS_A)],
                        out_hbm.at[c, pl.ds(base, ROWS_A)])

    @pl.when(s >= SPLIT_TILE)
    def _writeout_b():
        pltpu.sync_copy(agg_sh.at[pl.ds(base, ROWS_B)],
                        out_hbm.at[c, pl.ds(base, ROWS_B)])


_sc_agg = pl.kernel(
    _sc_agg_body,
    out_type=jax.ShapeDtypeStruct((NC, N, D_IN), jnp.float32),
    mesh=plsc.VectorSubcoreMesh(core_axis_name="c", subcore_axis_name="s"),
    compiler_params=pltpu.CompilerParams(needs_layout_passes=False),
    scratch_types=(
        [pltpu.VMEM((CHUNK,), jnp.int32) for _ in range(NMETA)]
        + [pltpu.VMEM((CHUNK,), jnp.int32) for _ in range(NMETA)]
        + [pltpu.VMEM((CHUNK,), jnp.float32) for _ in range(NMETA)]
        + [pltpu.VMEM((CHUNK, D_IN), jnp.float32) for _ in range(NBUF)]
        + [pltpu.VMEM_SHARED((N, D_IN), jnp.float32)]
        + [pltpu.SemaphoreType.DMA for _ in range(NBUF * 2 + NMETA)]
    ),
)


def _mlp_body(a_ref, x_ref, wrel_ref, wroot_ref, wh_ref, wout_ref,
              brel_ref, bh_ref, bout_ref, o_ref):
    agg = (a_ref[0] + a_ref[1]).astype(jnp.bfloat16)
    h = (jnp.dot(agg, wrel_ref[...], preferred_element_type=jnp.float32)
         + jnp.dot(x_ref[...], wroot_ref[...],
                   preferred_element_type=jnp.float32)
         + brel_ref[...])
    h = jnp.maximum(h, 0.0).astype(jnp.bfloat16)
    h2 = jnp.dot(h, wh_ref[...], preferred_element_type=jnp.float32) + bh_ref[...]
    h2 = jnp.maximum(h2, 0.0) + jnp.log1p(jnp.exp(-jnp.abs(h2)))
    o_ref[...] = (jnp.dot(h2.astype(jnp.bfloat16), wout_ref[...],
                          preferred_element_type=jnp.float32)
                  + bout_ref[...])


_BLK = 2000


def _mlp(agg2, x, wrel_t, wroot_t, wh_t, wout_t, brel, bh, bout):
    grid = (N // _BLK,)
    return pl.pallas_call(
        _mlp_body,
        grid=grid,
        in_specs=[
            pl.BlockSpec((NC, _BLK, D_IN), lambda i: (0, i, 0)),
            pl.BlockSpec((_BLK, D_IN), lambda i: (i, 0)),
            pl.BlockSpec((D_IN, D_H), lambda i: (0, 0)),
            pl.BlockSpec((D_IN, D_H), lambda i: (0, 0)),
            pl.BlockSpec((D_H, D_H), lambda i: (0, 0)),
            pl.BlockSpec((D_H, D_OUT), lambda i: (0, 0)),
            pl.BlockSpec((1, D_H), lambda i: (0, 0)),
            pl.BlockSpec((1, D_H), lambda i: (0, 0)),
            pl.BlockSpec((1, D_OUT), lambda i: (0, 0)),
        ],
        out_specs=pl.BlockSpec((_BLK, D_OUT), lambda i: (i, 0)),
        out_shape=jax.ShapeDtypeStruct((N, D_OUT), jnp.float32),
    )(agg2, x, wrel_t, wroot_t, wh_t, wout_t, brel, bh, bout)


def kernel(feature_data, edge_info, edge_weights, W_rel, b_rel, W_root,
           W_h, b_h, W_out, b_out):
    ei = edge_info.astype(jnp.int32)
    agg2 = _sc_agg(feature_data, ei[0], ei[1], edge_weights)
    bf = jnp.bfloat16
    return _mlp(agg2, feature_data.astype(bf), W_rel.T.astype(bf),
                W_root.T.astype(bf), W_h.T.astype(bf), W_out.T.astype(bf),
                b_rel[None, :], b_h[None, :], b_out[None, :])
